# baseline (speedup 1.0000x reference)
import functools

import numpy as np
import jax
import jax.numpy as jnp
from jax import lax
from jax.experimental import pallas as pl
from jax.experimental.pallas import tpu as pltpu
from jax.experimental.pallas import tpu_sc as plsc

H = 128
CNT = 16
WID = H + CNT
_COLS_LO = np.arange(H).reshape(H // 32, 2, 16)[:, 0, :].reshape(-1)
_COLS_HI = np.arange(H).reshape(H // 32, 2, 16)[:, 1, :].reshape(-1)


def _node_proj_body(node_ref, wn_ref, g_ref, wg_ref, b_ref, out_ref):
    gvec = jnp.dot(g_ref[...], wg_ref[...], preferred_element_type=jnp.float32,
                   precision=lax.Precision.HIGHEST)
    out_ref[...] = (
        jnp.dot(node_ref[...], wn_ref[...], preferred_element_type=jnp.float32,
                precision=lax.Precision.HIGHEST)
        + gvec + b_ref[...]
    )


def _bf16_bits(x):
    u = jax.lax.bitcast_convert_type(x, jnp.uint32)
    return (u + jnp.uint32(0x7FFF) + ((u >> 16) & jnp.uint32(1))) >> 16


def _edge_part_body(es_ref, wl_ref, wh_ref, out_ref):
    hp = lambda a, b: jnp.dot(a, b, preferred_element_type=jnp.float32,
                              precision=lax.Precision.HIGHEST)
    lo = _bf16_bits(hp(es_ref[...], wl_ref[...]))
    hi = _bf16_bits(hp(es_ref[...], wh_ref[...]))
    out_ref[...] = jax.lax.bitcast_convert_type(lo | (hi << 16), jnp.int32)


def _sc_aggregate(nproj, epart, src, dst):
    info = plsc.get_sparse_core_info()
    nc, ns = info.num_cores, info.num_subcores
    nw = nc * ns
    n = nproj.shape[0]
    e = epart.shape[0]
    ew = e // nw
    ch = 48
    nchunk = ew // ch
    tail = ew - nchunk * ch
    rpt = n // ns

    mesh = plsc.VectorSubcoreMesh(core_axis_name="c", subcore_axis_name="s")
    zinit = jnp.zeros((rpt, WID), jnp.float32)

    nscat = ch // 16

    def body(nproj_h, epart_h, src_h, dst_h, zin_h, out_h,
             sv0, sv1, dv0, dv1, rows0, rows1, ep0, ep1, hb0, hb1, acc,
             g0, g1, e0, e1, s0, s1, is0, is1, id0, id1):
        c = lax.axis_index("c")
        s = lax.axis_index("s")
        w = s * nc + c
        base = w * ew
        srcv = (sv0, sv1)
        dstv = (dv0, dv1)
        rowsb = (rows0, rows1)
        epb = (ep0, ep1)
        hbs = (hb0, hb1)
        gsem = (g0, g1)
        esem = (e0, e1)
        ssem = (s0, s1)
        is_sem = (is0, is1)
        id_sem = (id0, id1)

        pltpu.sync_copy(zin_h, acc.at[pl.ds(s * rpt, rpt)])
        onev = jnp.where(lax.iota(jnp.int32, 16) == 0, 1.0, 0.0)
        for i in range(ch):
            hb0[i, pl.ds(H, CNT)] = onev
            hb1[i, pl.ds(H, CNT)] = onev

        def issue_src(ci, b):
            pltpu.async_copy(src_h.at[pl.ds(base + ci * ch, ch)],
                             srcv[b], is_sem[b])

        def issue_dst(ci, b):
            pltpu.async_copy(dst_h.at[pl.ds(base + ci * ch, ch)],
                             dstv[b], id_sem[b])

        def issue_loads(ci, b):
            pltpu.async_copy(nproj_h.at[srcv[b]], rowsb[b], gsem[b])
            pltpu.async_copy(epart_h.at[pl.ds(base + ci * ch, ch)],
                             epb[b], esem[b])

        def wait(sem, src_ref, dst_ref):
            pltpu.make_async_copy(src_ref, dst_ref, sem).wait()

        def compute(b):
            @plsc.parallel_loop(0, ch, step=1, unroll=8)
            def _(i):
                for j2 in range(H // 32):
                    ev = epb[b][i, pl.ds(j2 * 16, 16)]
                    elo = jax.lax.bitcast_convert_type(ev << 16, jnp.float32)
                    ehi = jax.lax.bitcast_convert_type(
                        ev & jnp.int32(-65536), jnp.float32)
                    sl0 = pl.ds(j2 * 32, 16)
                    sl1 = pl.ds(j2 * 32 + 16, 16)
                    hbs[b][i, sl0] = jnp.maximum(rowsb[b][i, sl0] + elo, 0.0)
                    hbs[b][i, sl1] = jnp.maximum(rowsb[b][i, sl1] + ehi, 0.0)

        def issue_scatter(b):
            for k in range(nscat):
                dvec = dstv[b][pl.ds(k * 16, 16)]
                pltpu.async_copy(hbs[b].at[pl.ds(k * 16, 16)],
                                 acc.at[dvec], ssem[b], add=True)

        def wait_scatter(b):
            for k in range(nscat):
                dvec = dstv[b][pl.ds(k * 16, 16)]
                pltpu.make_async_copy(hbs[b].at[pl.ds(k * 16, 16)],
                                      acc.at[dvec], ssem[b]).wait()

        issue_src(0, 0)
        issue_src(1, 1)
        issue_dst(0, 0)
        issue_dst(1, 1)
        wait(is_sem[0], src_h.at[pl.ds(base, ch)], srcv[0])
        issue_loads(0, 0)
        plsc.subcore_barrier()

        def pair(pi, carry):
            for b in range(2):
                ci = 2 * pi + b
                nxt = 1 - b

                @pl.when(ci + 1 < nchunk)
                def _():
                    wait(is_sem[nxt],
                         src_h.at[pl.ds(base + (ci + 1) * ch, ch)], srcv[nxt])
                    issue_loads(ci + 1, nxt)

                wait(gsem[b], nproj_h.at[srcv[b]], rowsb[b])
                wait(esem[b], epart_h.at[pl.ds(base + ci * ch, ch)], epb[b])

                @pl.when(ci >= 2)
                def _():
                    wait_scatter(b)

                @pl.when(ci + 2 < nchunk)
                def _():
                    issue_src(ci + 2, b)

                compute(b)
                wait(id_sem[b], dst_h.at[pl.ds(base + ci * ch, ch)], dstv[b])
                issue_scatter(b)

                @pl.when(ci + 2 < nchunk)
                def _():
                    issue_dst(ci + 2, b)
            return carry

        lax.fori_loop(0, nchunk // 2, pair, 0)
        for b in range(2):
            wait_scatter(b)

        if tail:
            toff = base + nchunk * ch
            pltpu.sync_copy(src_h.at[pl.ds(toff, tail)],
                            sv0.at[pl.ds(0, tail)])
            pltpu.sync_copy(dst_h.at[pl.ds(toff, tail)],
                            dv0.at[pl.ds(0, tail)])
            pltpu.async_copy(nproj_h.at[sv0.at[pl.ds(0, tail)]],
                             rows0.at[pl.ds(0, tail)], g0).wait()
            pltpu.sync_copy(epart_h.at[pl.ds(toff, tail)],
                            ep0.at[pl.ds(0, tail)])
            for i in range(tail):
                for j2 in range(H // 32):
                    ev = jax.lax.bitcast_convert_type(
                        ep0[i, pl.ds(j2 * 16, 16)] << 16, jnp.float32)
                    evh = jax.lax.bitcast_convert_type(
                        ep0[i, pl.ds(j2 * 16, 16)] & jnp.int32(-65536), jnp.float32)
                    sl0 = pl.ds(j2 * 32, 16)
                    sl1 = pl.ds(j2 * 32 + 16, 16)
                    hb0[i, sl0] = jnp.maximum(rows0[i, sl0] + ev, 0.0)
                    hb0[i, sl1] = jnp.maximum(rows0[i, sl1] + evh, 0.0)
            for k in range(tail // 16):
                dvec = dv0[pl.ds(k * 16, 16)]
                pltpu.sync_copy(hb0.at[pl.ds(k * 16, 16)], acc.at[dvec],
                                add=True)

        plsc.subcore_barrier()
        pltpu.sync_copy(acc.at[pl.ds(s * rpt, rpt)],
                        out_h.at[c, pl.ds(s * rpt, rpt)])

    f = pl.kernel(
        body,
        out_type=jax.ShapeDtypeStruct((nc, n, WID), jnp.float32),
        mesh=mesh,
        compiler_params=pltpu.CompilerParams(use_tc_tiling_on_sc=False),
        scratch_types=[
            pltpu.VMEM((ch,), jnp.int32),
            pltpu.VMEM((ch,), jnp.int32),
            pltpu.VMEM((ch,), jnp.int32),
            pltpu.VMEM((ch,), jnp.int32),
            pltpu.VMEM((ch, H), jnp.float32),
            pltpu.VMEM((ch, H), jnp.float32),
            pltpu.VMEM((ch, H // 2), jnp.int32),
            pltpu.VMEM((ch, H // 2), jnp.int32),
            pltpu.VMEM((ch, WID), jnp.float32),
            pltpu.VMEM((ch, WID), jnp.float32),
            pltpu.VMEM_SHARED((n, WID), jnp.float32),
            pltpu.SemaphoreType.DMA,
            pltpu.SemaphoreType.DMA,
            pltpu.SemaphoreType.DMA,
            pltpu.SemaphoreType.DMA,
            pltpu.SemaphoreType.DMA,
            pltpu.SemaphoreType.DMA,
            pltpu.SemaphoreType.DMA,
            pltpu.SemaphoreType.DMA,
            pltpu.SemaphoreType.DMA,
            pltpu.SemaphoreType.DMA,
        ],
    )
    return f(nproj, epart, src, dst, zinit)


def _final_body(part_ref, node_ref, w2m_ref, b2m_ref, w1un_ref, w1ua_ref,
                g_ref, w1ug_ref, b1u_ref, w2u_ref, b2u_ref, gam_ref, bet_ref,
                out_ref):
    hp = lambda a, b: jnp.dot(a, b, preferred_element_type=jnp.float32,
                              precision=lax.Precision.HIGHEST)
    s = part_ref[0] + part_ref[1]
    agg = s[:, :H]
    deg = jnp.sum(s[:, H:], axis=1, keepdims=True)
    aggregated = (hp(agg, w2m_ref[...]) + deg * b2m_ref[...]) / jnp.maximum(deg, 1.0)
    ns = node_ref[...]
    gvec = hp(g_ref[...], w1ug_ref[...]) + b1u_ref[...]
    u = hp(ns, w1un_ref[...]) + hp(aggregated, w1ua_ref[...]) + gvec
    h2 = jnp.maximum(u, 0.0)
    x = ns + hp(h2, w2u_ref[...]) + b2u_ref[...]
    mu = jnp.mean(x, axis=1, keepdims=True)
    xc = x - mu
    var = jnp.mean(xc * xc, axis=1, keepdims=True)
    out_ref[...] = xc * lax.rsqrt(var + 1e-5) * gam_ref[...] + bet_ref[...]


def kernel(node_state, edge_index, edge_state, global_state,
           W1m, b1m, W2m, b2m, W1u, b1u, W2u, b2u, gamma, beta):
    n, h = node_state.shape
    e, ed = edge_state.shape
    g = global_state.shape[0]
    assert h == H

    src = edge_index[0]
    dst = edge_index[1]
    g_row = global_state[None, :]

    nproj = pl.pallas_call(
        _node_proj_body,
        out_shape=jax.ShapeDtypeStruct((n, H), jnp.float32),
    )(node_state, W1m[:h], g_row, W1m[h + ed:], b1m[None, :])

    eb = 8000
    epart = pl.pallas_call(
        _edge_part_body,
        grid=(e // eb,),
        in_specs=[
            pl.BlockSpec((eb, ed), lambda i: (i, 0)),
            pl.BlockSpec((ed, H // 2), lambda i: (0, 0)),
            pl.BlockSpec((ed, H // 2), lambda i: (0, 0)),
        ],
        out_specs=pl.BlockSpec((eb, H // 2), lambda i: (i, 0)),
        out_shape=jax.ShapeDtypeStruct((e, H // 2), jnp.int32),
    )(edge_state, W1m[h:h + ed][:, _COLS_LO], W1m[h:h + ed][:, _COLS_HI])

    partials = _sc_aggregate(nproj, epart, src, dst)

    nb = 2000
    out = pl.pallas_call(
        _final_body,
        grid=(n // nb,),
        in_specs=[
            pl.BlockSpec((2, nb, WID), lambda i: (0, i, 0)),
            pl.BlockSpec((nb, H), lambda i: (i, 0)),
            pl.BlockSpec((H, H), lambda i: (0, 0)),
            pl.BlockSpec((1, H), lambda i: (0, 0)),
            pl.BlockSpec((H, H), lambda i: (0, 0)),
            pl.BlockSpec((H, H), lambda i: (0, 0)),
            pl.BlockSpec((1, g), lambda i: (0, 0)),
            pl.BlockSpec((g, H), lambda i: (0, 0)),
            pl.BlockSpec((1, H), lambda i: (0, 0)),
            pl.BlockSpec((H, H), lambda i: (0, 0)),
            pl.BlockSpec((1, H), lambda i: (0, 0)),
            pl.BlockSpec((1, H), lambda i: (0, 0)),
            pl.BlockSpec((1, H), lambda i: (0, 0)),
        ],
        out_specs=pl.BlockSpec((nb, H), lambda i: (i, 0)),
        out_shape=jax.ShapeDtypeStruct((n, H), jnp.float32),
    )(partials, node_state, W2m, b2m[None, :], W1u[:h], W1u[h:2 * h],
      g_row, W1u[2 * h:], b1u[None, :], W2u, b2u[None, :],
      gamma[None, :], beta[None, :])

    return out


# speedup vs baseline: 1.4980x; 1.4980x over previous
import functools

import numpy as np
import jax
import jax.numpy as jnp
from jax import lax
from jax.experimental import pallas as pl
from jax.experimental.pallas import tpu as pltpu
from jax.experimental.pallas import tpu_sc as plsc

H = 128
CNT = 16
WID = H + CNT


def _node_proj_body(node_ref, wn_ref, g_ref, wg_ref, b_ref, out_ref):
    gvec = jnp.dot(g_ref[...], wg_ref[...], preferred_element_type=jnp.float32,
                   precision=lax.Precision.HIGHEST)
    out_ref[...] = (
        jnp.dot(node_ref[...], wn_ref[...], preferred_element_type=jnp.float32,
                precision=lax.Precision.HIGHEST)
        + gvec + b_ref[...]
    )


def _edge_part_body(es_ref, we_ref, out_ref):
    out_ref[...] = jnp.dot(es_ref[...], we_ref[...],
                           preferred_element_type=jnp.float32,
                           precision=lax.Precision.HIGHEST)


def _sc_aggregate(nproj, epart, src, dst):
    info = plsc.get_sparse_core_info()
    nc, ns = info.num_cores, info.num_subcores
    nw = nc * ns
    n = nproj.shape[0]
    e = epart.shape[0]
    ew = e // nw
    ch = 48
    nchunk = ew // ch
    tail = ew - nchunk * ch
    rpt = n // ns

    mesh = plsc.VectorSubcoreMesh(core_axis_name="c", subcore_axis_name="s")
    zinit = jnp.zeros((rpt, WID), jnp.float32)

    nscat = ch // 16

    def body(nproj_h, epart_h, src_h, dst_h, zin_h, out_h,
             sv0, sv1, dv0, dv1, rows0, rows1, ep0, ep1, hb0, hb1, acc,
             g0, g1, e0, e1, s0, s1, is0, is1, id0, id1):
        c = lax.axis_index("c")
        s = lax.axis_index("s")
        w = s * nc + c
        base = w * ew
        srcv = (sv0, sv1)
        dstv = (dv0, dv1)
        rowsb = (rows0, rows1)
        epb = (ep0, ep1)
        hbs = (hb0, hb1)
        gsem = (g0, g1)
        esem = (e0, e1)
        ssem = (s0, s1)
        is_sem = (is0, is1)
        id_sem = (id0, id1)

        pltpu.sync_copy(zin_h, acc.at[pl.ds(s * rpt, rpt)])
        onev = jnp.where(lax.iota(jnp.int32, 16) == 0, 1.0, 0.0)
        for i in range(ch):
            hb0[i, pl.ds(H, CNT)] = onev
            hb1[i, pl.ds(H, CNT)] = onev

        def issue_src(ci, b):
            pltpu.async_copy(src_h.at[pl.ds(base + ci * ch, ch)],
                             srcv[b], is_sem[b])

        def issue_dst(ci, b):
            pltpu.async_copy(dst_h.at[pl.ds(base + ci * ch, ch)],
                             dstv[b], id_sem[b])

        def issue_loads(ci, b):
            pltpu.async_copy(nproj_h.at[srcv[b]], rowsb[b], gsem[b])
            pltpu.async_copy(epart_h.at[pl.ds(base + ci * ch, ch)],
                             epb[b], esem[b])

        def wait(sem, src_ref, dst_ref):
            pltpu.make_async_copy(src_ref, dst_ref, sem).wait()

        def compute(b):
            @plsc.parallel_loop(0, ch, step=1, unroll=4)
            def _(i):
                for j in range(H // 16):
                    sl = pl.ds(j * 16, 16)
                    hbs[b][i, sl] = jnp.maximum(
                        rowsb[b][i, sl] + epb[b][i, sl], 0.0)

        def issue_scatter(b):
            for k in range(nscat):
                dvec = dstv[b][pl.ds(k * 16, 16)]
                pltpu.async_copy(hbs[b].at[pl.ds(k * 16, 16)],
                                 acc.at[dvec], ssem[b], add=True)

        def wait_scatter(b):
            for k in range(nscat):
                dvec = dstv[b][pl.ds(k * 16, 16)]
                pltpu.make_async_copy(hbs[b].at[pl.ds(k * 16, 16)],
                                      acc.at[dvec], ssem[b]).wait()

        issue_src(0, 0)
        issue_src(1, 1)
        issue_dst(0, 0)
        issue_dst(1, 1)
        wait(is_sem[0], src_h.at[pl.ds(base, ch)], srcv[0])
        issue_loads(0, 0)
        plsc.subcore_barrier()

        def pair(pi, carry):
            for b in range(2):
                ci = 2 * pi + b
                nxt = 1 - b

                @pl.when(ci + 1 < nchunk)
                def _():
                    wait(is_sem[nxt],
                         src_h.at[pl.ds(base + (ci + 1) * ch, ch)], srcv[nxt])
                    issue_loads(ci + 1, nxt)

                wait(gsem[b], nproj_h.at[srcv[b]], rowsb[b])
                wait(esem[b], epart_h.at[pl.ds(base + ci * ch, ch)], epb[b])

                @pl.when(ci >= 2)
                def _():
                    wait_scatter(b)

                @pl.when(ci + 2 < nchunk)
                def _():
                    issue_src(ci + 2, b)

                compute(b)
                wait(id_sem[b], dst_h.at[pl.ds(base + ci * ch, ch)], dstv[b])
                issue_scatter(b)

                @pl.when(ci + 2 < nchunk)
                def _():
                    issue_dst(ci + 2, b)
            return carry

        lax.fori_loop(0, nchunk // 2, pair, 0)
        for b in range(2):
            wait_scatter(b)

        if tail:
            toff = base + nchunk * ch
            pltpu.sync_copy(src_h.at[pl.ds(toff, tail)],
                            sv0.at[pl.ds(0, tail)])
            pltpu.sync_copy(dst_h.at[pl.ds(toff, tail)],
                            dv0.at[pl.ds(0, tail)])
            pltpu.async_copy(nproj_h.at[sv0.at[pl.ds(0, tail)]],
                             rows0.at[pl.ds(0, tail)], g0).wait()
            pltpu.sync_copy(epart_h.at[pl.ds(toff, tail)],
                            ep0.at[pl.ds(0, tail)])
            for i in range(tail):
                for j in range(H // 16):
                    sl = pl.ds(j * 16, 16)
                    hb0[i, sl] = jnp.maximum(rows0[i, sl] + ep0[i, sl], 0.0)
            for k in range(tail // 16):
                dvec = dv0[pl.ds(k * 16, 16)]
                pltpu.sync_copy(hb0.at[pl.ds(k * 16, 16)], acc.at[dvec],
                                add=True)

        plsc.subcore_barrier()
        pltpu.sync_copy(acc.at[pl.ds(s * rpt, rpt)],
                        out_h.at[c, pl.ds(s * rpt, rpt)])

    f = pl.kernel(
        body,
        out_type=jax.ShapeDtypeStruct((nc, n, WID), jnp.float32),
        mesh=mesh,
        compiler_params=pltpu.CompilerParams(use_tc_tiling_on_sc=False),
        scratch_types=[
            pltpu.VMEM((ch,), jnp.int32),
            pltpu.VMEM((ch,), jnp.int32),
            pltpu.VMEM((ch,), jnp.int32),
            pltpu.VMEM((ch,), jnp.int32),
            pltpu.VMEM((ch, H), jnp.float32),
            pltpu.VMEM((ch, H), jnp.float32),
            pltpu.VMEM((ch, H), jnp.float32),
            pltpu.VMEM((ch, H), jnp.float32),
            pltpu.VMEM((ch, WID), jnp.float32),
            pltpu.VMEM((ch, WID), jnp.float32),
            pltpu.VMEM_SHARED((n, WID), jnp.float32),
            pltpu.SemaphoreType.DMA,
            pltpu.SemaphoreType.DMA,
            pltpu.SemaphoreType.DMA,
            pltpu.SemaphoreType.DMA,
            pltpu.SemaphoreType.DMA,
            pltpu.SemaphoreType.DMA,
            pltpu.SemaphoreType.DMA,
            pltpu.SemaphoreType.DMA,
            pltpu.SemaphoreType.DMA,
            pltpu.SemaphoreType.DMA,
        ],
    )
    return f(nproj, epart, src, dst, zinit)


def _final_body(part_ref, node_ref, w2m_ref, b2m_ref, w1un_ref, w1ua_ref,
                g_ref, w1ug_ref, b1u_ref, w2u_ref, b2u_ref, gam_ref, bet_ref,
                out_ref):
    hp = lambda a, b: jnp.dot(a, b, preferred_element_type=jnp.float32,
                              precision=lax.Precision.HIGHEST)
    s = part_ref[0] + part_ref[1]
    agg = s[:, :H]
    deg = jnp.sum(s[:, H:], axis=1, keepdims=True)
    aggregated = (hp(agg, w2m_ref[...]) + deg * b2m_ref[...]) / jnp.maximum(deg, 1.0)
    ns = node_ref[...]
    gvec = hp(g_ref[...], w1ug_ref[...]) + b1u_ref[...]
    u = hp(ns, w1un_ref[...]) + hp(aggregated, w1ua_ref[...]) + gvec
    h2 = jnp.maximum(u, 0.0)
    x = ns + hp(h2, w2u_ref[...]) + b2u_ref[...]
    mu = jnp.mean(x, axis=1, keepdims=True)
    xc = x - mu
    var = jnp.mean(xc * xc, axis=1, keepdims=True)
    out_ref[...] = xc * lax.rsqrt(var + 1e-5) * gam_ref[...] + bet_ref[...]


def kernel(node_state, edge_index, edge_state, global_state,
           W1m, b1m, W2m, b2m, W1u, b1u, W2u, b2u, gamma, beta):
    n, h = node_state.shape
    e, ed = edge_state.shape
    g = global_state.shape[0]
    assert h == H

    src = edge_index[0]
    dst = edge_index[1]
    g_row = global_state[None, :]

    nproj = pl.pallas_call(
        _node_proj_body,
        out_shape=jax.ShapeDtypeStruct((n, H), jnp.float32),
    )(node_state, W1m[:h], g_row, W1m[h + ed:], b1m[None, :])

    eb = 8000
    epart = pl.pallas_call(
        _edge_part_body,
        grid=(e // eb,),
        in_specs=[
            pl.BlockSpec((eb, ed), lambda i: (i, 0)),
            pl.BlockSpec((ed, H), lambda i: (0, 0)),
        ],
        out_specs=pl.BlockSpec((eb, H), lambda i: (i, 0)),
        out_shape=jax.ShapeDtypeStruct((e, H), jnp.float32),
    )(edge_state, W1m[h:h + ed])

    partials = _sc_aggregate(nproj, epart, src, dst)

    nb = 2000
    out = pl.pallas_call(
        _final_body,
        grid=(n // nb,),
        in_specs=[
            pl.BlockSpec((2, nb, WID), lambda i: (0, i, 0)),
            pl.BlockSpec((nb, H), lambda i: (i, 0)),
            pl.BlockSpec((H, H), lambda i: (0, 0)),
            pl.BlockSpec((1, H), lambda i: (0, 0)),
            pl.BlockSpec((H, H), lambda i: (0, 0)),
            pl.BlockSpec((H, H), lambda i: (0, 0)),
            pl.BlockSpec((1, g), lambda i: (0, 0)),
            pl.BlockSpec((g, H), lambda i: (0, 0)),
            pl.BlockSpec((1, H), lambda i: (0, 0)),
            pl.BlockSpec((H, H), lambda i: (0, 0)),
            pl.BlockSpec((1, H), lambda i: (0, 0)),
            pl.BlockSpec((1, H), lambda i: (0, 0)),
            pl.BlockSpec((1, H), lambda i: (0, 0)),
        ],
        out_specs=pl.BlockSpec((nb, H), lambda i: (i, 0)),
        out_shape=jax.ShapeDtypeStruct((n, H), jnp.float32),
    )(partials, node_state, W2m, b2m[None, :], W1u[:h], W1u[h:2 * h],
      g_row, W1u[2 * h:], b1u[None, :], W2u, b2u[None, :],
      gamma[None, :], beta[None, :])

    return out
